# baseline (speedup 1.0000x reference)
import functools

import jax
import jax.numpy as jnp
from jax import lax
from jax.experimental import pallas as pl
from jax.experimental.pallas import tpu as pltpu
from jax.experimental.pallas import tpu_sc as plsc

N = 10000
D = 128
E_EDGES = 320000
N_STEPS = 4
ETA = 0.1
DT = 0.1
LN_EPS = 1e-5

NC = 2
NS = 16
NW = NC * NS
EPW = E_EDGES // NW
CHUNK = 128
NCHF = EPW // CHUNK
REM = EPW - NCHF * CHUNK
ZB = 624
ZL = N - (NS - 1) * ZB

BLK = 1000
NB = N // BLK


def _zsplit(sid, fn):
    @pl.when(sid < NS - 1)
    def _():
        fn(pl.ds(0, ZB), pl.ds(sid * ZB, ZB))

    @pl.when(sid == NS - 1)
    def _():
        fn(pl.ds(0, ZL), pl.ds((NS - 1) * ZB, ZL))


def _sc_body(table, g1, s1, zrows, acc_out,
             gi_v, sc0, sc1, sc16, r0, r1, r16, acc_sh, gs0, gs1, is0, is1):
    cid = lax.axis_index("c")
    sid = lax.axis_index("s")
    wid = sid * NC + cid
    e0 = wid * EPW

    _zsplit(sid, lambda zs, sl: pltpu.sync_copy(zrows.at[zs], acc_sh.at[sl]))
    pltpu.sync_copy(g1.at[pl.ds(e0, EPW)], gi_v)
    plsc.subcore_barrier()

    def iload(i, scv, sem):
        return pltpu.async_copy(s1.at[pl.ds(e0 + i * CHUNK, CHUNK)], scv, sem)

    def iwait(i, scv, sem):
        pltpu.make_async_copy(s1.at[pl.ds(e0 + i * CHUNK, CHUNK)], scv,
                              sem).wait()

    def gath(i, rbuf, sem):
        return pltpu.async_copy(
            table.at[gi_v.at[pl.ds(i * CHUNK, CHUNK)]], rbuf, sem)

    def gath_wait(i, rbuf, sem):
        pltpu.make_async_copy(
            table.at[gi_v.at[pl.ds(i * CHUNK, CHUNK)]], rbuf, sem).wait()

    iload(0, sc0, is0)
    iload(1, sc1, is1)
    gath(0, r0, gs0)

    def pair(g, carry):
        i0 = 2 * g
        i1 = i0 + 1
        gath(i1, r1, gs1)
        gath_wait(i0, r0, gs0)
        iwait(i0, sc0, is0)
        pltpu.sync_copy(r0, acc_sh.at[sc0], add=True)

        @pl.when(i0 + 2 < NCHF)
        def _():
            iload(i0 + 2, sc0, is0)
            gath(i0 + 2, r0, gs0)

        gath_wait(i1, r1, gs1)
        iwait(i1, sc1, is1)
        pltpu.sync_copy(r1, acc_sh.at[sc1], add=True)

        @pl.when(i1 + 2 < NCHF)
        def _():
            iload(i1 + 2, sc1, is1)

        return carry

    lax.fori_loop(0, NCHF // 2, pair, 0)
    if REM:
        toff = e0 + NCHF * CHUNK
        pltpu.sync_copy(s1.at[pl.ds(toff, REM)], sc16)
        pltpu.sync_copy(table.at[gi_v.at[pl.ds(NCHF * CHUNK, REM)]], r16)
        pltpu.sync_copy(r16, acc_sh.at[sc16], add=True)

    plsc.subcore_barrier()
    _zsplit(sid, lambda zs, sl: pltpu.sync_copy(acc_sh.at[sl],
                                                acc_out.at[cid, sl]))


def _deg_body(ones_h, s1, zrows, acc_out,
              sc0, sc1, sc16, r0, acc_sh, is0, is1):
    cid = lax.axis_index("c")
    sid = lax.axis_index("s")
    wid = sid * NC + cid
    e0 = wid * EPW

    _zsplit(sid, lambda zs, sl: pltpu.sync_copy(zrows.at[zs], acc_sh.at[sl]))
    pltpu.sync_copy(ones_h, r0)
    plsc.subcore_barrier()

    def iload(i, scv, sem):
        return pltpu.async_copy(s1.at[pl.ds(e0 + i * CHUNK, CHUNK)], scv, sem)

    def iwait(i, scv, sem):
        pltpu.make_async_copy(s1.at[pl.ds(e0 + i * CHUNK, CHUNK)], scv,
                              sem).wait()

    iload(0, sc0, is0)
    iload(1, sc1, is1)

    def pair(g, carry):
        i0 = 2 * g
        i1 = i0 + 1
        iwait(i0, sc0, is0)
        pltpu.sync_copy(r0, acc_sh.at[sc0], add=True)

        @pl.when(i0 + 2 < NCHF)
        def _():
            iload(i0 + 2, sc0, is0)

        iwait(i1, sc1, is1)
        pltpu.sync_copy(r0, acc_sh.at[sc1], add=True)

        @pl.when(i1 + 2 < NCHF)
        def _():
            iload(i1 + 2, sc1, is1)

        return carry

    lax.fori_loop(0, NCHF // 2, pair, 0)
    if REM:
        pltpu.sync_copy(s1.at[pl.ds(e0 + NCHF * CHUNK, REM)], sc16)
        pltpu.sync_copy(r0.at[pl.ds(0, REM)], acc_sh.at[sc16], add=True)

    plsc.subcore_barrier()
    _zsplit(sid, lambda zs, sl: pltpu.sync_copy(acc_sh.at[sl],
                                                acc_out.at[cid, sl]))


@functools.lru_cache(maxsize=None)
def _make_sc_degrees():
    mesh = plsc.VectorSubcoreMesh(core_axis_name="c", subcore_axis_name="s",
                                  num_cores=NC, num_subcores=NS)
    return pl.kernel(
        _deg_body,
        out_type=jax.ShapeDtypeStruct((NC, N, D), jnp.float32),
        mesh=mesh,
        scratch_types=[
            pltpu.VMEM((CHUNK,), jnp.int32),
            pltpu.VMEM((CHUNK,), jnp.int32),
            pltpu.VMEM((REM,), jnp.int32),
            pltpu.VMEM((CHUNK, D), jnp.float32),
            pltpu.VMEM_SHARED((N, D), jnp.float32),
            pltpu.SemaphoreType.DMA,
            pltpu.SemaphoreType.DMA,
        ],
    )


def _sc_degrees(*args):
    return _make_sc_degrees()(*args)


@functools.lru_cache(maxsize=None)
def _make_sc_scatter():
    mesh = plsc.VectorSubcoreMesh(core_axis_name="c", subcore_axis_name="s",
                                  num_cores=NC, num_subcores=NS)
    return pl.kernel(
        _sc_body,
        out_type=jax.ShapeDtypeStruct((NC, N, D), jnp.float32),
        mesh=mesh,
        scratch_types=[
            pltpu.VMEM((EPW,), jnp.int32),
            pltpu.VMEM((CHUNK,), jnp.int32),
            pltpu.VMEM((CHUNK,), jnp.int32),
            pltpu.VMEM((REM,), jnp.int32),
            pltpu.VMEM((CHUNK, D), jnp.float32),
            pltpu.VMEM((CHUNK, D), jnp.float32),
            pltpu.VMEM((REM, D), jnp.float32),
            pltpu.VMEM_SHARED((N, D), jnp.float32),
            pltpu.SemaphoreType.DMA,
            pltpu.SemaphoreType.DMA,
            pltpu.SemaphoreType.DMA,
            pltpu.SemaphoreType.DMA,
        ],
    )


def _sc_scatter(*args):
    return _make_sc_scatter()(*args)


def _dotT(a, b):
    return lax.dot_general(a, b, (((1,), (1,)), ((), ())),
                           preferred_element_type=jnp.float32)


def _mlp(mu, Wp1, bp1, Wp2, bp2):
    t = jnp.tanh(_dotT(mu, Wp1) + bp1)
    return _dotT(t, Wp2) + bp2


def _init_body(x_ref, W1_ref, b1_ref, g_ref, be_ref, obs_ref,
               Wp1_ref, bp1_ref, Wp2_ref, bp2_ref, mu_ref, p_ref):
    x = x_ref[...]
    h = _dotT(x, W1_ref[...]) + b1_ref[...]
    m = jnp.mean(h, axis=1, keepdims=True)
    v = jnp.mean((h - m) ** 2, axis=1, keepdims=True)
    h = (h - m) / jnp.sqrt(v + LN_EPS) * g_ref[...] + be_ref[...]
    h = 0.5 * h * (1.0 + lax.erf(h / jnp.sqrt(2.0).astype(jnp.float32)))
    mu = jnp.where(obs_ref[...] > 0, x, h)
    mu_ref[...] = mu
    p_ref[...] = _mlp(mu, Wp1_ref[...], bp1_ref[...], Wp2_ref[...], bp2_ref[...])


def _eps_body(mu_ref, x_ref, p0_ref, p1_ref, c0_ref, c1_ref, obs_ref,
              We_ref, bee_ref, eps_ref, ew_ref):
    counts = jnp.mean(c0_ref[...] + c1_ref[...], axis=1, keepdims=True)
    denom = jnp.where(counts > 0, counts, 1.0)
    pred = (p0_ref[...] + p1_ref[...]) / denom
    mu = mu_ref[...]
    eps = mu - pred
    root = (counts == 0) & (obs_ref[...] == 0)
    eps = jnp.where(root, mu, eps)
    eps_ref[...] = eps
    ew_ref[...] = _dotT(eps, We_ref[...]) + bee_ref[...]


def _mu_body(mu_ref, x_ref, s0_ref, s1_ref, lp_ref, obs_ref, eps_ref,
             Wp1_ref, bp1_ref, Wp2_ref, bp2_ref, mun_ref, p_ref):
    err = s0_ref[...] + s1_ref[...]
    prec = jax.nn.softplus(lp_ref[...])
    mu = mu_ref[...] + ETA * (-prec * eps_ref[...] + err)
    mu = jnp.where(obs_ref[...] > 0, x_ref[...], mu)
    mun_ref[...] = mu
    p_ref[...] = _mlp(mu, Wp1_ref[...], bp1_ref[...], Wp2_ref[...], bp2_ref[...])


def _eps_last_body(mu_ref, x_ref, p0_ref, p1_ref, c0_ref, c1_ref, obs_ref,
                   We_ref, bee_ref, ph_ref, lp_ref,
                   eps_ref, ew_ref, u_ref, v_ref, fe_ref):
    i = pl.program_id(0)

    @pl.when(i == 0)
    def _():
        u_ref[...] = jnp.zeros_like(u_ref)
        v_ref[...] = jnp.zeros_like(v_ref)
        fe_ref[...] = jnp.zeros_like(fe_ref)

    counts = jnp.mean(c0_ref[...] + c1_ref[...], axis=1, keepdims=True)
    denom = jnp.where(counts > 0, counts, 1.0)
    pred = (p0_ref[...] + p1_ref[...]) / denom
    mu = mu_ref[...]
    eps = mu - pred
    root = (counts == 0) & (obs_ref[...] == 0)
    eps = jnp.where(root, mu, eps)
    eps_ref[...] = eps
    ew_ref[...] = _dotT(eps, We_ref[...]) + bee_ref[...]
    nrm = jnp.sqrt(jnp.sum(eps * eps, axis=1, keepdims=True))
    en = eps / (nrm + 1e-8)
    ph = ph_ref[...]
    u_ref[...] += jnp.sum(en * jnp.sin(ph), axis=0, keepdims=True)
    v_ref[...] += jnp.sum(en * jnp.cos(ph), axis=0, keepdims=True)
    prec = jax.nn.softplus(lp_ref[...])
    fe_ref[...] += 0.5 * jnp.sum(prec * eps * eps).reshape(1, 1)


def _mu_last_body(mu_ref, x_ref, s0_ref, s1_ref, lp_ref, obs_ref, eps_ref,
                  ph_ref, u_ref, v_ref, mun_ref, pn_ref, cc_ref, ss_ref):
    i = pl.program_id(0)

    @pl.when(i == 0)
    def _():
        cc_ref[...] = jnp.zeros_like(cc_ref)
        ss_ref[...] = jnp.zeros_like(ss_ref)

    err = s0_ref[...] + s1_ref[...]
    prec = jax.nn.softplus(lp_ref[...])
    eps = eps_ref[...]
    mu = mu_ref[...] + ETA * (-prec * eps + err)
    mun_ref[...] = jnp.where(obs_ref[...] > 0, x_ref[...], mu)
    nrm = jnp.sqrt(jnp.sum(eps * eps, axis=1, keepdims=True))
    en = eps / (nrm + 1e-8)
    Ws = jnp.sum(en * u_ref[...], axis=1, keepdims=True) * 0.01
    Wc = jnp.sum(en * v_ref[...], axis=1, keepdims=True) * 0.01
    ph = ph_ref[...]
    pn = ph + DT * (jnp.cos(ph) * Ws - jnp.sin(ph) * Wc)
    pn_ref[...] = pn
    cc_ref[...] += jnp.sum(jnp.cos(pn)).reshape(1, 1)
    ss_ref[...] += jnp.sum(jnp.sin(pn)).reshape(1, 1)


def _rows(i):
    return (i, 0)


def _full(i):
    return (0, 0)


_ROW = pl.BlockSpec((BLK, D), _rows)
_ROW1 = pl.BlockSpec((BLK, 1), _rows)
_ROW16 = pl.BlockSpec((BLK, 16), _rows)
_WMAT = pl.BlockSpec((D, D), _full)
_WVEC = pl.BlockSpec((1, D), _full)
_SCAL = pl.BlockSpec((1, 1), _full)

_ND = jax.ShapeDtypeStruct((N, D), jnp.float32)
_N1 = jax.ShapeDtypeStruct((N, 1), jnp.float32)
_1D = jax.ShapeDtypeStruct((1, D), jnp.float32)
_11 = jax.ShapeDtypeStruct((1, 1), jnp.float32)

_tc_init = pl.pallas_call(
    _init_body, grid=(NB,),
    in_specs=[_ROW, _WMAT, _WVEC, _WVEC, _WVEC, _ROW1, _WMAT, _WVEC, _WMAT, _WVEC],
    out_specs=[_ROW, _ROW],
    out_shape=[_ND, _ND],
)

_tc_eps = pl.pallas_call(
    _eps_body, grid=(NB,),
    in_specs=[_ROW, _ROW, _ROW, _ROW, _ROW, _ROW, _ROW1, _WMAT, _WVEC],
    out_specs=[_ROW, _ROW],
    out_shape=[_ND, _ND],
)

_tc_mu = pl.pallas_call(
    _mu_body, grid=(NB,),
    in_specs=[_ROW, _ROW, _ROW, _ROW, _ROW1, _ROW1, _ROW,
              _WMAT, _WVEC, _WMAT, _WVEC],
    out_specs=[_ROW, _ROW],
    out_shape=[_ND, _ND],
)

_tc_eps_last = pl.pallas_call(
    _eps_last_body, grid=(NB,),
    in_specs=[_ROW, _ROW, _ROW, _ROW, _ROW, _ROW, _ROW1, _WMAT, _WVEC,
              _ROW1, _ROW1],
    out_specs=[_ROW, _ROW, _WVEC, _WVEC, _SCAL],
    out_shape=[_ND, _ND, _1D, _1D, _11],
)

_tc_mu_last = pl.pallas_call(
    _mu_last_body, grid=(NB,),
    in_specs=[_ROW, _ROW, _ROW, _ROW, _ROW1, _ROW1, _ROW, _ROW1,
              _WVEC, _WVEC],
    out_specs=[_ROW, _ROW1, _SCAL, _SCAL],
    out_shape=[_ND, _N1, _11, _11],
)


def kernel(x, edge_index, obs_mask, edge_type, W1, b1, gamma, beta,
           Wp1, bp1, Wp2, bp2, We, be, log_precision, phases):
    del edge_type
    src1 = edge_index[0]
    dst1 = edge_index[1]
    obsf = obs_mask.astype(jnp.float32).reshape(N, 1)
    lp2 = log_precision.reshape(N, 1)
    ph2 = phases.reshape(N, 1)
    zrows = jnp.zeros((ZL, D), jnp.float32)
    ones_ch = jnp.ones((CHUNK, D), jnp.float32)
    b1r = b1.reshape(1, D)
    gr = gamma.reshape(1, D)
    btr = beta.reshape(1, D)
    bp1r = bp1.reshape(1, D)
    bp2r = bp2.reshape(1, D)
    ber = be.reshape(1, D)

    mu, P = _tc_init(x, W1, b1r, gr, btr, obsf, Wp1, bp1r, Wp2, bp2r)
    cnt = _sc_degrees(ones_ch, dst1, zrows)
    c0, c1 = cnt[0], cnt[1]

    for step in range(N_STEPS - 1):
        pred = _sc_scatter(P, src1, dst1, zrows)
        eps, ew = _tc_eps(mu, x, pred[0], pred[1], c0, c1, obsf, We, ber)
        S = _sc_scatter(ew, dst1, src1, zrows)
        mu, P = _tc_mu(mu, x, S[0], S[1], lp2, obsf, eps,
                       Wp1, bp1r, Wp2, bp2r)

    pred = _sc_scatter(P, src1, dst1, zrows)
    eps, ew, u, v, fe = _tc_eps_last(mu, x, pred[0], pred[1], c0, c1, obsf,
                                     We, ber, ph2, lp2)
    S = _sc_scatter(ew, dst1, src1, zrows)
    mu, pn2, cc, ss = _tc_mu_last(mu, x, S[0], S[1], lp2, obsf, eps,
                                  ph2, u, v)
    free_energy = fe[0, 0]
    order_param = jnp.sqrt((cc[0, 0] / N) ** 2 + (ss[0, 0] / N) ** 2)
    return mu, eps, free_energy, order_param, pn2.reshape(N)


# speedup vs baseline: 1.0462x; 1.0462x over previous
import functools

import jax
import jax.numpy as jnp
from jax import lax
from jax.experimental import pallas as pl
from jax.experimental.pallas import tpu as pltpu
from jax.experimental.pallas import tpu_sc as plsc

N = 10000
D = 128
E_EDGES = 320000
N_STEPS = 4
ETA = 0.1
DT = 0.1
LN_EPS = 1e-5

NC = 2
NS = 16
NW = NC * NS
EPW = E_EDGES // NW
CHUNK = 128
NCHF = EPW // CHUNK
REM = EPW - NCHF * CHUNK
ZB = 624
ZL = N - (NS - 1) * ZB

BLK = 2000
NB = N // BLK


def _zsplit(sid, fn):
    @pl.when(sid < NS - 1)
    def _():
        fn(pl.ds(0, ZB), pl.ds(sid * ZB, ZB))

    @pl.when(sid == NS - 1)
    def _():
        fn(pl.ds(0, ZL), pl.ds((NS - 1) * ZB, ZL))


def _sc_body(table, g1, s1, zrows, acc_out,
             gi_v, sc0, sc1, sc16, r0, r1, r16, acc_sh, gs0, gs1, is0, is1):
    cid = lax.axis_index("c")
    sid = lax.axis_index("s")
    wid = sid * NC + cid
    e0 = wid * EPW

    _zsplit(sid, lambda zs, sl: pltpu.sync_copy(zrows.at[zs], acc_sh.at[sl]))
    pltpu.sync_copy(g1.at[pl.ds(e0, EPW)], gi_v)
    plsc.subcore_barrier()

    def iload(i, scv, sem):
        return pltpu.async_copy(s1.at[pl.ds(e0 + i * CHUNK, CHUNK)], scv, sem)

    def iwait(i, scv, sem):
        pltpu.make_async_copy(s1.at[pl.ds(e0 + i * CHUNK, CHUNK)], scv,
                              sem).wait()

    def gath(i, rbuf, sem):
        return pltpu.async_copy(
            table.at[gi_v.at[pl.ds(i * CHUNK, CHUNK)]], rbuf, sem)

    def gath_wait(i, rbuf, sem):
        pltpu.make_async_copy(
            table.at[gi_v.at[pl.ds(i * CHUNK, CHUNK)]], rbuf, sem).wait()

    iload(0, sc0, is0)
    iload(1, sc1, is1)
    gath(0, r0, gs0)

    def pair(g, carry):
        i0 = 2 * g
        i1 = i0 + 1
        gath(i1, r1, gs1)
        gath_wait(i0, r0, gs0)
        iwait(i0, sc0, is0)
        pltpu.sync_copy(r0, acc_sh.at[sc0], add=True)

        @pl.when(i0 + 2 < NCHF)
        def _():
            iload(i0 + 2, sc0, is0)
            gath(i0 + 2, r0, gs0)

        gath_wait(i1, r1, gs1)
        iwait(i1, sc1, is1)
        pltpu.sync_copy(r1, acc_sh.at[sc1], add=True)

        @pl.when(i1 + 2 < NCHF)
        def _():
            iload(i1 + 2, sc1, is1)

        return carry

    lax.fori_loop(0, NCHF // 2, pair, 0)
    if REM:
        toff = e0 + NCHF * CHUNK
        pltpu.sync_copy(s1.at[pl.ds(toff, REM)], sc16)
        pltpu.sync_copy(table.at[gi_v.at[pl.ds(NCHF * CHUNK, REM)]], r16)
        pltpu.sync_copy(r16, acc_sh.at[sc16], add=True)

    plsc.subcore_barrier()
    _zsplit(sid, lambda zs, sl: pltpu.sync_copy(acc_sh.at[sl],
                                                acc_out.at[cid, sl]))


def _deg_body(ones_h, s1, zrows, acc_out,
              sc0, sc1, sc16, r0, acc_sh, is0, is1):
    cid = lax.axis_index("c")
    sid = lax.axis_index("s")
    wid = sid * NC + cid
    e0 = wid * EPW

    _zsplit(sid, lambda zs, sl: pltpu.sync_copy(zrows.at[zs], acc_sh.at[sl]))
    pltpu.sync_copy(ones_h, r0)
    plsc.subcore_barrier()

    def iload(i, scv, sem):
        return pltpu.async_copy(s1.at[pl.ds(e0 + i * CHUNK, CHUNK)], scv, sem)

    def iwait(i, scv, sem):
        pltpu.make_async_copy(s1.at[pl.ds(e0 + i * CHUNK, CHUNK)], scv,
                              sem).wait()

    iload(0, sc0, is0)
    iload(1, sc1, is1)

    def pair(g, carry):
        i0 = 2 * g
        i1 = i0 + 1
        iwait(i0, sc0, is0)
        pltpu.sync_copy(r0, acc_sh.at[sc0], add=True)

        @pl.when(i0 + 2 < NCHF)
        def _():
            iload(i0 + 2, sc0, is0)

        iwait(i1, sc1, is1)
        pltpu.sync_copy(r0, acc_sh.at[sc1], add=True)

        @pl.when(i1 + 2 < NCHF)
        def _():
            iload(i1 + 2, sc1, is1)

        return carry

    lax.fori_loop(0, NCHF // 2, pair, 0)
    if REM:
        pltpu.sync_copy(s1.at[pl.ds(e0 + NCHF * CHUNK, REM)], sc16)
        pltpu.sync_copy(r0.at[pl.ds(0, REM)], acc_sh.at[sc16], add=True)

    plsc.subcore_barrier()
    _zsplit(sid, lambda zs, sl: pltpu.sync_copy(acc_sh.at[sl],
                                                acc_out.at[cid, sl]))


@functools.lru_cache(maxsize=None)
def _make_sc_degrees():
    mesh = plsc.VectorSubcoreMesh(core_axis_name="c", subcore_axis_name="s",
                                  num_cores=NC, num_subcores=NS)
    return pl.kernel(
        _deg_body,
        out_type=jax.ShapeDtypeStruct((NC, N, D), jnp.float32),
        mesh=mesh,
        scratch_types=[
            pltpu.VMEM((CHUNK,), jnp.int32),
            pltpu.VMEM((CHUNK,), jnp.int32),
            pltpu.VMEM((REM,), jnp.int32),
            pltpu.VMEM((CHUNK, D), jnp.float32),
            pltpu.VMEM_SHARED((N, D), jnp.float32),
            pltpu.SemaphoreType.DMA,
            pltpu.SemaphoreType.DMA,
        ],
    )


def _sc_degrees(*args):
    return _make_sc_degrees()(*args)


@functools.lru_cache(maxsize=None)
def _make_sc_scatter():
    mesh = plsc.VectorSubcoreMesh(core_axis_name="c", subcore_axis_name="s",
                                  num_cores=NC, num_subcores=NS)
    return pl.kernel(
        _sc_body,
        out_type=jax.ShapeDtypeStruct((NC, N, D), jnp.float32),
        mesh=mesh,
        scratch_types=[
            pltpu.VMEM((EPW,), jnp.int32),
            pltpu.VMEM((CHUNK,), jnp.int32),
            pltpu.VMEM((CHUNK,), jnp.int32),
            pltpu.VMEM((REM,), jnp.int32),
            pltpu.VMEM((CHUNK, D), jnp.float32),
            pltpu.VMEM((CHUNK, D), jnp.float32),
            pltpu.VMEM((REM, D), jnp.float32),
            pltpu.VMEM_SHARED((N, D), jnp.float32),
            pltpu.SemaphoreType.DMA,
            pltpu.SemaphoreType.DMA,
            pltpu.SemaphoreType.DMA,
            pltpu.SemaphoreType.DMA,
        ],
    )


def _sc_scatter(*args):
    return _make_sc_scatter()(*args)


def _dotT(a, b):
    return lax.dot_general(a, b, (((1,), (1,)), ((), ())),
                           preferred_element_type=jnp.float32)


def _mlp(mu, Wp1, bp1, Wp2, bp2):
    t = jnp.tanh(_dotT(mu, Wp1) + bp1)
    return _dotT(t, Wp2) + bp2


def _init_body(x_ref, W1_ref, b1_ref, g_ref, be_ref, obs_ref,
               Wp1_ref, bp1_ref, Wp2_ref, bp2_ref, mu_ref, p_ref):
    x = x_ref[...]
    h = _dotT(x, W1_ref[...]) + b1_ref[...]
    m = jnp.mean(h, axis=1, keepdims=True)
    v = jnp.mean((h - m) ** 2, axis=1, keepdims=True)
    h = (h - m) / jnp.sqrt(v + LN_EPS) * g_ref[...] + be_ref[...]
    h = 0.5 * h * (1.0 + lax.erf(h / jnp.sqrt(2.0).astype(jnp.float32)))
    mu = jnp.where(obs_ref[...] > 0, x, h)
    mu_ref[...] = mu
    p_ref[...] = _mlp(mu, Wp1_ref[...], bp1_ref[...], Wp2_ref[...], bp2_ref[...])


def _eps_body(mu_ref, x_ref, p0_ref, p1_ref, c0_ref, c1_ref, obs_ref,
              We_ref, bee_ref, eps_ref, ew_ref):
    counts = jnp.mean(c0_ref[...] + c1_ref[...], axis=1, keepdims=True)
    denom = jnp.where(counts > 0, counts, 1.0)
    pred = (p0_ref[...] + p1_ref[...]) / denom
    mu = mu_ref[...]
    eps = mu - pred
    root = (counts == 0) & (obs_ref[...] == 0)
    eps = jnp.where(root, mu, eps)
    eps_ref[...] = eps
    ew_ref[...] = _dotT(eps, We_ref[...]) + bee_ref[...]


def _mu_body(mu_ref, x_ref, s0_ref, s1_ref, lp_ref, obs_ref, eps_ref,
             Wp1_ref, bp1_ref, Wp2_ref, bp2_ref, mun_ref, p_ref):
    err = s0_ref[...] + s1_ref[...]
    prec = jax.nn.softplus(lp_ref[...])
    mu = mu_ref[...] + ETA * (-prec * eps_ref[...] + err)
    mu = jnp.where(obs_ref[...] > 0, x_ref[...], mu)
    mun_ref[...] = mu
    p_ref[...] = _mlp(mu, Wp1_ref[...], bp1_ref[...], Wp2_ref[...], bp2_ref[...])


def _phase1_body(eps_ref, ph_ref, lp_ref, u_ref, v_ref, fe_ref):
    i = pl.program_id(0)

    @pl.when(i == 0)
    def _():
        u_ref[...] = jnp.zeros_like(u_ref)
        v_ref[...] = jnp.zeros_like(v_ref)
        fe_ref[...] = jnp.zeros_like(fe_ref)

    eps = eps_ref[...]
    nrm = jnp.sqrt(jnp.sum(eps * eps, axis=1, keepdims=True))
    en = eps / (nrm + 1e-8)
    ph = ph_ref[...]
    st = jnp.sin(ph)
    ct = jnp.cos(ph)
    u_ref[...] += jnp.sum(en * st, axis=0, keepdims=True)
    v_ref[...] += jnp.sum(en * ct, axis=0, keepdims=True)
    prec = jax.nn.softplus(lp_ref[...])
    fe_ref[...] += 0.5 * jnp.sum(prec * eps * eps).reshape(1, 1)


def _phase2_body(eps_ref, ph_ref, u_ref, v_ref, pn_ref, cc_ref, ss_ref):
    i = pl.program_id(0)

    @pl.when(i == 0)
    def _():
        cc_ref[...] = jnp.zeros_like(cc_ref)
        ss_ref[...] = jnp.zeros_like(ss_ref)

    eps = eps_ref[...]
    nrm = jnp.sqrt(jnp.sum(eps * eps, axis=1, keepdims=True))
    en = eps / (nrm + 1e-8)
    Ws = jnp.sum(en * u_ref[...], axis=1, keepdims=True) * 0.01
    Wc = jnp.sum(en * v_ref[...], axis=1, keepdims=True) * 0.01
    ph = ph_ref[...]
    st = jnp.sin(ph)
    ct = jnp.cos(ph)
    pn = ph + DT * (ct * Ws - st * Wc)
    pn_ref[...] = pn
    cc_ref[...] += jnp.sum(jnp.cos(pn)).reshape(1, 1)
    ss_ref[...] += jnp.sum(jnp.sin(pn)).reshape(1, 1)


def _rows(i):
    return (i, 0)


def _full(i):
    return (0, 0)


_ROW = pl.BlockSpec((BLK, D), _rows)
_ROW1 = pl.BlockSpec((BLK, 1), _rows)
_ROW16 = pl.BlockSpec((BLK, 16), _rows)
_WMAT = pl.BlockSpec((D, D), _full)
_WVEC = pl.BlockSpec((1, D), _full)
_SCAL = pl.BlockSpec((1, 1), _full)

_ND = jax.ShapeDtypeStruct((N, D), jnp.float32)
_N1 = jax.ShapeDtypeStruct((N, 1), jnp.float32)
_1D = jax.ShapeDtypeStruct((1, D), jnp.float32)
_11 = jax.ShapeDtypeStruct((1, 1), jnp.float32)

_tc_init = pl.pallas_call(
    _init_body, grid=(NB,),
    in_specs=[_ROW, _WMAT, _WVEC, _WVEC, _WVEC, _ROW1, _WMAT, _WVEC, _WMAT, _WVEC],
    out_specs=[_ROW, _ROW],
    out_shape=[_ND, _ND],
)

_tc_eps = pl.pallas_call(
    _eps_body, grid=(NB,),
    in_specs=[_ROW, _ROW, _ROW, _ROW, _ROW, _ROW, _ROW1, _WMAT, _WVEC],
    out_specs=[_ROW, _ROW],
    out_shape=[_ND, _ND],
)

_tc_mu = pl.pallas_call(
    _mu_body, grid=(NB,),
    in_specs=[_ROW, _ROW, _ROW, _ROW, _ROW1, _ROW1, _ROW,
              _WMAT, _WVEC, _WMAT, _WVEC],
    out_specs=[_ROW, _ROW],
    out_shape=[_ND, _ND],
)

_tc_phase1 = pl.pallas_call(
    _phase1_body, grid=(NB,),
    in_specs=[_ROW, _ROW1, _ROW1],
    out_specs=[_WVEC, _WVEC, _SCAL],
    out_shape=[_1D, _1D, _11],
)

_tc_phase2 = pl.pallas_call(
    _phase2_body, grid=(NB,),
    in_specs=[_ROW, _ROW1, _WVEC, _WVEC],
    out_specs=[_ROW1, _SCAL, _SCAL],
    out_shape=[_N1, _11, _11],
)


def kernel(x, edge_index, obs_mask, edge_type, W1, b1, gamma, beta,
           Wp1, bp1, Wp2, bp2, We, be, log_precision, phases):
    del edge_type
    src1 = edge_index[0]
    dst1 = edge_index[1]
    obsf = obs_mask.astype(jnp.float32).reshape(N, 1)
    lp2 = log_precision.reshape(N, 1)
    ph2 = phases.reshape(N, 1)
    zrows = jnp.zeros((ZL, D), jnp.float32)
    ones_ch = jnp.ones((CHUNK, D), jnp.float32)
    b1r = b1.reshape(1, D)
    gr = gamma.reshape(1, D)
    btr = beta.reshape(1, D)
    bp1r = bp1.reshape(1, D)
    bp2r = bp2.reshape(1, D)
    ber = be.reshape(1, D)

    mu, P = _tc_init(x, W1, b1r, gr, btr, obsf, Wp1, bp1r, Wp2, bp2r)
    cnt = _sc_degrees(ones_ch, dst1, zrows)
    c0, c1 = cnt[0], cnt[1]

    eps = None
    for step in range(N_STEPS):
        pred = _sc_scatter(P, src1, dst1, zrows)
        eps, ew = _tc_eps(mu, x, pred[0], pred[1], c0, c1, obsf, We, ber)
        S = _sc_scatter(ew, dst1, src1, zrows)
        mu, P = _tc_mu(mu, x, S[0], S[1], lp2, obsf, eps,
                       Wp1, bp1r, Wp2, bp2r)

    u, v, fe = _tc_phase1(eps, ph2, lp2)
    pn2, cc, ss = _tc_phase2(eps, ph2, u, v)
    free_energy = fe[0, 0]
    order_param = jnp.sqrt((cc[0, 0] / N) ** 2 + (ss[0, 0] / N) ** 2)
    return mu, eps, free_energy, order_param, pn2.reshape(N)


# speedup vs baseline: 1.0478x; 1.0015x over previous
import functools

import jax
import jax.numpy as jnp
from jax import lax
from jax.experimental import pallas as pl
from jax.experimental.pallas import tpu as pltpu
from jax.experimental.pallas import tpu_sc as plsc

N = 10000
D = 128
E_EDGES = 320000
N_STEPS = 4
ETA = 0.1
DT = 0.1
LN_EPS = 1e-5

NC = 2
NS = 16
NW = NC * NS
EPW = E_EDGES // NW
CHUNK = 128
NCHF = EPW // CHUNK
REM = EPW - NCHF * CHUNK
ZB = 624
ZL = N - (NS - 1) * ZB

BLK = 2000
NB = N // BLK


def _zsplit(sid, fn):
    @pl.when(sid < NS - 1)
    def _():
        fn(pl.ds(0, ZB), pl.ds(sid * ZB, ZB))

    @pl.when(sid == NS - 1)
    def _():
        fn(pl.ds(0, ZL), pl.ds((NS - 1) * ZB, ZL))


def _sc_body(table, g1, s1, zrows, acc_out,
             gi_v, sc0, sc1, sc16, r0, r1, r16, acc_sh, gs0, gs1, is0, is1):
    cid = lax.axis_index("c")
    sid = lax.axis_index("s")
    wid = sid * NC + cid
    e0 = wid * EPW

    _zsplit(sid, lambda zs, sl: pltpu.sync_copy(zrows.at[zs], acc_sh.at[sl]))
    pltpu.sync_copy(g1.at[pl.ds(e0, EPW)], gi_v)
    plsc.subcore_barrier()

    def iload(i, scv, sem):
        return pltpu.async_copy(s1.at[pl.ds(e0 + i * CHUNK, CHUNK)], scv, sem)

    def iwait(i, scv, sem):
        pltpu.make_async_copy(s1.at[pl.ds(e0 + i * CHUNK, CHUNK)], scv,
                              sem).wait()

    def gath(i, rbuf, sem):
        return pltpu.async_copy(
            table.at[gi_v.at[pl.ds(i * CHUNK, CHUNK)]], rbuf, sem)

    def gath_wait(i, rbuf, sem):
        pltpu.make_async_copy(
            table.at[gi_v.at[pl.ds(i * CHUNK, CHUNK)]], rbuf, sem).wait()

    iload(0, sc0, is0)
    iload(1, sc1, is1)
    gath(0, r0, gs0)

    def pair(g, carry):
        i0 = 2 * g
        i1 = i0 + 1
        gath(i1, r1, gs1)
        gath_wait(i0, r0, gs0)
        iwait(i0, sc0, is0)
        pltpu.sync_copy(r0, acc_sh.at[sc0], add=True)

        @pl.when(i0 + 2 < NCHF)
        def _():
            iload(i0 + 2, sc0, is0)
            gath(i0 + 2, r0, gs0)

        gath_wait(i1, r1, gs1)
        iwait(i1, sc1, is1)
        pltpu.sync_copy(r1, acc_sh.at[sc1], add=True)

        @pl.when(i1 + 2 < NCHF)
        def _():
            iload(i1 + 2, sc1, is1)

        return carry

    lax.fori_loop(0, NCHF // 2, pair, 0)
    if REM:
        toff = e0 + NCHF * CHUNK
        pltpu.sync_copy(s1.at[pl.ds(toff, REM)], sc16)
        pltpu.sync_copy(table.at[gi_v.at[pl.ds(NCHF * CHUNK, REM)]], r16)
        pltpu.sync_copy(r16, acc_sh.at[sc16], add=True)

    plsc.subcore_barrier()
    _zsplit(sid, lambda zs, sl: pltpu.sync_copy(acc_sh.at[sl],
                                                acc_out.at[cid, sl]))


def _deg_body(ones_h, s1, zrows, acc_out,
              sc0, sc1, sc16, r0, acc_sh, is0, is1):
    cid = lax.axis_index("c")
    sid = lax.axis_index("s")
    wid = sid * NC + cid
    e0 = wid * EPW

    _zsplit(sid, lambda zs, sl: pltpu.sync_copy(zrows.at[zs], acc_sh.at[sl]))
    pltpu.sync_copy(ones_h, r0)
    plsc.subcore_barrier()

    def iload(i, scv, sem):
        return pltpu.async_copy(s1.at[pl.ds(e0 + i * CHUNK, CHUNK)], scv, sem)

    def iwait(i, scv, sem):
        pltpu.make_async_copy(s1.at[pl.ds(e0 + i * CHUNK, CHUNK)], scv,
                              sem).wait()

    iload(0, sc0, is0)
    iload(1, sc1, is1)

    def pair(g, carry):
        i0 = 2 * g
        i1 = i0 + 1
        iwait(i0, sc0, is0)
        pltpu.sync_copy(r0, acc_sh.at[sc0], add=True)

        @pl.when(i0 + 2 < NCHF)
        def _():
            iload(i0 + 2, sc0, is0)

        iwait(i1, sc1, is1)
        pltpu.sync_copy(r0, acc_sh.at[sc1], add=True)

        @pl.when(i1 + 2 < NCHF)
        def _():
            iload(i1 + 2, sc1, is1)

        return carry

    lax.fori_loop(0, NCHF // 2, pair, 0)
    if REM:
        pltpu.sync_copy(s1.at[pl.ds(e0 + NCHF * CHUNK, REM)], sc16)
        pltpu.sync_copy(r0.at[pl.ds(0, REM)], acc_sh.at[sc16], add=True)

    plsc.subcore_barrier()
    _zsplit(sid, lambda zs, sl: pltpu.sync_copy(acc_sh.at[sl],
                                                acc_out.at[cid, sl]))


@functools.lru_cache(maxsize=None)
def _make_sc_degrees():
    mesh = plsc.VectorSubcoreMesh(core_axis_name="c", subcore_axis_name="s",
                                  num_cores=NC, num_subcores=NS)
    return pl.kernel(
        _deg_body,
        out_type=jax.ShapeDtypeStruct((NC, N, D), jnp.float32),
        mesh=mesh,
        scratch_types=[
            pltpu.VMEM((CHUNK,), jnp.int32),
            pltpu.VMEM((CHUNK,), jnp.int32),
            pltpu.VMEM((REM,), jnp.int32),
            pltpu.VMEM((CHUNK, D), jnp.float32),
            pltpu.VMEM_SHARED((N, D), jnp.float32),
            pltpu.SemaphoreType.DMA,
            pltpu.SemaphoreType.DMA,
        ],
    )


def _sc_degrees(*args):
    return _make_sc_degrees()(*args)


@functools.lru_cache(maxsize=None)
def _make_sc_scatter():
    mesh = plsc.VectorSubcoreMesh(core_axis_name="c", subcore_axis_name="s",
                                  num_cores=NC, num_subcores=NS)
    return pl.kernel(
        _sc_body,
        out_type=jax.ShapeDtypeStruct((NC, N, D), jnp.float32),
        mesh=mesh,
        scratch_types=[
            pltpu.VMEM((EPW,), jnp.int32),
            pltpu.VMEM((CHUNK,), jnp.int32),
            pltpu.VMEM((CHUNK,), jnp.int32),
            pltpu.VMEM((REM,), jnp.int32),
            pltpu.VMEM((CHUNK, D), jnp.float32),
            pltpu.VMEM((CHUNK, D), jnp.float32),
            pltpu.VMEM((REM, D), jnp.float32),
            pltpu.VMEM_SHARED((N, D), jnp.float32),
            pltpu.SemaphoreType.DMA,
            pltpu.SemaphoreType.DMA,
            pltpu.SemaphoreType.DMA,
            pltpu.SemaphoreType.DMA,
        ],
    )


def _sc_scatter(*args):
    return _make_sc_scatter()(*args)


def _dotT(a, b):
    return lax.dot_general(a, b, (((1,), (1,)), ((), ())),
                           preferred_element_type=jnp.float32)


def _mlp(mu, Wp1, bp1, Wp2, bp2):
    t = jnp.tanh(_dotT(mu, Wp1) + bp1)
    return _dotT(t, Wp2) + bp2


def _init_body(x_ref, W1_ref, b1_ref, g_ref, be_ref, obs_ref,
               Wp1_ref, bp1_ref, Wp2_ref, bp2_ref, mu_ref, p_ref):
    x = x_ref[...]
    h = _dotT(x, W1_ref[...]) + b1_ref[...]
    m = jnp.mean(h, axis=1, keepdims=True)
    v = jnp.mean((h - m) ** 2, axis=1, keepdims=True)
    h = (h - m) / jnp.sqrt(v + LN_EPS) * g_ref[...] + be_ref[...]
    h = 0.5 * h * (1.0 + lax.erf(h / jnp.sqrt(2.0).astype(jnp.float32)))
    mu = jnp.where(obs_ref[...] > 0, x, h)
    mu_ref[...] = mu
    p_ref[...] = _mlp(mu, Wp1_ref[...], bp1_ref[...], Wp2_ref[...], bp2_ref[...])


def _eps_body(mu_ref, x_ref, p0_ref, p1_ref, c0_ref, c1_ref, obs_ref,
              We_ref, bee_ref, eps_ref, ew_ref):
    counts = jnp.mean(c0_ref[...] + c1_ref[...], axis=1, keepdims=True)
    denom = jnp.where(counts > 0, counts, 1.0)
    pred = (p0_ref[...] + p1_ref[...]) / denom
    mu = mu_ref[...]
    eps = mu - pred
    root = (counts == 0) & (obs_ref[...] == 0)
    eps = jnp.where(root, mu, eps)
    eps_ref[...] = eps
    ew_ref[...] = _dotT(eps, We_ref[...]) + bee_ref[...]


def _mu_body(mu_ref, x_ref, s0_ref, s1_ref, lp_ref, obs_ref, eps_ref,
             Wp1_ref, bp1_ref, Wp2_ref, bp2_ref, mun_ref, p_ref):
    err = s0_ref[...] + s1_ref[...]
    prec = jax.nn.softplus(lp_ref[...])
    mu = mu_ref[...] + ETA * (-prec * eps_ref[...] + err)
    mu = jnp.where(obs_ref[...] > 0, x_ref[...], mu)
    mun_ref[...] = mu
    p_ref[...] = _mlp(mu, Wp1_ref[...], bp1_ref[...], Wp2_ref[...], bp2_ref[...])


def _phase1_body(eps_ref, ph_ref, lp_ref, u_ref, v_ref, fe_ref):
    i = pl.program_id(0)

    @pl.when(i == 0)
    def _():
        u_ref[...] = jnp.zeros_like(u_ref)
        v_ref[...] = jnp.zeros_like(v_ref)
        fe_ref[...] = jnp.zeros_like(fe_ref)

    eps = eps_ref[...]
    nrm = jnp.sqrt(jnp.sum(eps * eps, axis=1, keepdims=True))
    en = eps / (nrm + 1e-8)
    ph = ph_ref[...]
    st = jnp.sin(ph)
    ct = jnp.cos(ph)
    u_ref[...] += jnp.sum(en * st, axis=0, keepdims=True)
    v_ref[...] += jnp.sum(en * ct, axis=0, keepdims=True)
    prec = jax.nn.softplus(lp_ref[...])
    fe_ref[...] += 0.5 * jnp.sum(prec * eps * eps).reshape(1, 1)


def _phase2_body(eps_ref, ph_ref, u_ref, v_ref, pn_ref, cc_ref, ss_ref):
    i = pl.program_id(0)

    @pl.when(i == 0)
    def _():
        cc_ref[...] = jnp.zeros_like(cc_ref)
        ss_ref[...] = jnp.zeros_like(ss_ref)

    eps = eps_ref[...]
    nrm = jnp.sqrt(jnp.sum(eps * eps, axis=1, keepdims=True))
    en = eps / (nrm + 1e-8)
    Ws = jnp.sum(en * u_ref[...], axis=1, keepdims=True) * 0.01
    Wc = jnp.sum(en * v_ref[...], axis=1, keepdims=True) * 0.01
    ph = ph_ref[...]
    st = jnp.sin(ph)
    ct = jnp.cos(ph)
    pn = ph + DT * (ct * Ws - st * Wc)
    pn_ref[...] = pn
    cc_ref[...] += jnp.sum(jnp.cos(pn)).reshape(1, 1)
    ss_ref[...] += jnp.sum(jnp.sin(pn)).reshape(1, 1)


def _rows(i):
    return (i, 0)


def _full(i):
    return (0, 0)


_ROW = pl.BlockSpec((BLK, D), _rows)
_ROW1 = pl.BlockSpec((BLK, 1), _rows)
_ROW16 = pl.BlockSpec((BLK, 16), _rows)
_WMAT = pl.BlockSpec((D, D), _full)
_WVEC = pl.BlockSpec((1, D), _full)
_SCAL = pl.BlockSpec((1, 1), _full)

_ND = jax.ShapeDtypeStruct((N, D), jnp.float32)
_N1 = jax.ShapeDtypeStruct((N, 1), jnp.float32)
_1D = jax.ShapeDtypeStruct((1, D), jnp.float32)
_11 = jax.ShapeDtypeStruct((1, 1), jnp.float32)

_tc_init = pl.pallas_call(
    _init_body, grid=(NB,),
    in_specs=[_ROW, _WMAT, _WVEC, _WVEC, _WVEC, _ROW1, _WMAT, _WVEC, _WMAT, _WVEC],
    out_specs=[_ROW, _ROW],
    out_shape=[_ND, _ND],
)

_tc_eps = pl.pallas_call(
    _eps_body, grid=(NB,),
    in_specs=[_ROW, _ROW, _ROW, _ROW, _ROW, _ROW, _ROW1, _WMAT, _WVEC],
    out_specs=[_ROW, _ROW],
    out_shape=[_ND, _ND],
)

_tc_mu = pl.pallas_call(
    _mu_body, grid=(NB,),
    in_specs=[_ROW, _ROW, _ROW, _ROW, _ROW1, _ROW1, _ROW,
              _WMAT, _WVEC, _WMAT, _WVEC],
    out_specs=[_ROW, _ROW],
    out_shape=[_ND, _ND],
)

_tc_phase1 = pl.pallas_call(
    _phase1_body, grid=(NB,),
    in_specs=[_ROW, _ROW1, _ROW1],
    out_specs=[_WVEC, _WVEC, _SCAL],
    out_shape=[_1D, _1D, _11],
)

_tc_phase2 = pl.pallas_call(
    _phase2_body, grid=(NB,),
    in_specs=[_ROW, _ROW1, _WVEC, _WVEC],
    out_specs=[_ROW1, _SCAL, _SCAL],
    out_shape=[_N1, _11, _11],
)


def kernel(x, edge_index, obs_mask, edge_type, W1, b1, gamma, beta,
           Wp1, bp1, Wp2, bp2, We, be, log_precision, phases):
    del edge_type
    src1 = edge_index[0]
    dst1 = edge_index[1]
    obsf = obs_mask.astype(jnp.float32).reshape(N, 1)
    lp2 = log_precision.reshape(N, 1)
    ph2 = phases.reshape(N, 1)
    zrows = jnp.zeros((ZL, D), jnp.float32)
    ones_ch = jnp.ones((CHUNK, D), jnp.float32)
    b1r = b1.reshape(1, D)
    gr = gamma.reshape(1, D)
    btr = beta.reshape(1, D)
    bp1r = bp1.reshape(1, D)
    bp2r = bp2.reshape(1, D)
    ber = be.reshape(1, D)

    cnt = _sc_degrees(ones_ch, dst1, zrows)
    c0, c1 = cnt[0], cnt[1]
    mu, P = _tc_init(x, W1, b1r, gr, btr, obsf, Wp1, bp1r, Wp2, bp2r)

    eps = None
    for step in range(N_STEPS):
        pred = _sc_scatter(P, src1, dst1, zrows)
        eps, ew = _tc_eps(mu, x, pred[0], pred[1], c0, c1, obsf, We, ber)
        S = _sc_scatter(ew, dst1, src1, zrows)
        mu, P = _tc_mu(mu, x, S[0], S[1], lp2, obsf, eps,
                       Wp1, bp1r, Wp2, bp2r)

    u, v, fe = _tc_phase1(eps, ph2, lp2)
    pn2, cc, ss = _tc_phase2(eps, ph2, u, v)
    free_energy = fe[0, 0]
    order_param = jnp.sqrt((cc[0, 0] / N) ** 2 + (ss[0, 0] / N) ** 2)
    return mu, eps, free_energy, order_param, pn2.reshape(N)
